# baseline (speedup 1.0000x reference)
import functools

import jax
import jax.numpy as jnp
from jax import lax
from jax.experimental import pallas as pl
from jax.experimental.pallas import tpu as pltpu
from jax.experimental.pallas import tpu_sc as plsc

_EPS = 1e-5
_LANES = 16


def _rsqrt_newton(x):
    i = lax.bitcast_convert_type(x, jnp.int32)
    i = jnp.int32(0x5F3759DF) - lax.shift_right_arithmetic(i, 1)
    y = lax.bitcast_convert_type(i, jnp.float32)
    for _ in range(4):
        y = y * (1.5 - 0.5 * x * y * y)
    return y


def kernel(input_ids, position_ids, token_type_ids, word_table, pos_table,
           type_table, gamma, beta):
    B, S = input_ids.shape
    V, D = word_table.shape
    N = B * S
    ids_w = input_ids.reshape(N).astype(jnp.int32)
    ids_p = position_ids.reshape(N).astype(jnp.int32)
    ids_t = token_type_ids.reshape(N).astype(jnp.int32)

    info = plsc.get_sparse_core_info()
    NC, NS = info.num_cores, info.num_subcores
    NW = NC * NS
    K = 32
    G = K // _LANES
    per_w = N // NW
    n_chunks = per_w // K
    n_vec = D // _LANES
    n_grp = per_w // _LANES

    mesh = plsc.VectorSubcoreMesh(core_axis_name="c", subcore_axis_name="s")

    @functools.partial(
        pl.kernel,
        mesh=mesh,
        out_type=jax.ShapeDtypeStruct((N, D), jnp.float32),
        compiler_params=pltpu.CompilerParams(needs_layout_passes=False),
        scratch_types=[
            pltpu.VMEM((per_w,), jnp.int32),
            pltpu.VMEM((per_w,), jnp.int32),
            pltpu.VMEM((per_w,), jnp.int32),
            pltpu.VMEM((K, D), jnp.float32),
            pltpu.VMEM((K, D), jnp.float32),
            pltpu.VMEM((K, D), jnp.float32),
            pltpu.VMEM((K, D), jnp.float32),
            pltpu.VMEM((2, D), jnp.float32),
            pltpu.VMEM((D,), jnp.float32),
            pltpu.VMEM((D,), jnp.float32),
            pltpu.SMEM((per_w,), jnp.int32),
            pltpu.SMEM((K,), jnp.float32),
            pltpu.SMEM((K,), jnp.float32),
            pltpu.SemaphoreType.DMA,
            pltpu.SemaphoreType.DMA,
            pltpu.SemaphoreType.DMA,
            pltpu.SemaphoreType.DMA,
        ],
    )
    def emb_kernel(idsw_hbm, idsp_hbm, idst_hbm, word_hbm, pos_hbm, type_hbm,
                   gamma_hbm, beta_hbm, out_hbm,
                   idxw, idxp, idxt, bwA, bpA, bwB, bpB,
                   typev, gv, bv, stt, sinv, snm2,
                   semA, semB, semWA, semWB):
        wid = lax.axis_index("s") * NC + lax.axis_index("c")
        base = wid * per_w
        pltpu.sync_copy(gamma_hbm, gv)
        pltpu.sync_copy(beta_hbm, bv)
        pltpu.sync_copy(type_hbm, typev)
        pltpu.sync_copy(idsw_hbm.at[pl.ds(base, per_w)], idxw)
        pltpu.sync_copy(idsp_hbm.at[pl.ds(base, per_w)], idxp)
        pltpu.sync_copy(idst_hbm.at[pl.ds(base, per_w)], idxt)

        for g in range(n_grp):
            vec = idxt[pl.ds(g * _LANES, _LANES)]
            for u in range(_LANES):
                stt[g * _LANES + u] = vec[u]

        def fire(c, bw, bp, sem):
            off = pl.multiple_of(c * K, K)
            pltpu.async_copy(word_hbm.at[idxw.at[pl.ds(off, K)]], bw, sem)
            pltpu.async_copy(pos_hbm.at[idxp.at[pl.ds(off, K)]], bp, sem)

        def drain(bw, bp, sem):
            pltpu.make_async_copy(word_hbm.at[idxw.at[pl.ds(0, K)]], bw,
                                  sem).wait()
            pltpu.make_async_copy(pos_hbm.at[idxp.at[pl.ds(0, K)]], bp,
                                  sem).wait()

        def wb_wait(bw, semW):
            pltpu.make_async_copy(bw, out_hbm.at[pl.ds(0, K)], semW).wait()

        def compute_store(c, bw, bp, semW):
            coff = c * K

            def token_body(i, _):
                tt = stt[coff + i]
                ss = [jnp.zeros((_LANES,), jnp.float32) for _ in range(4)]
                qq = [jnp.zeros((_LANES,), jnp.float32) for _ in range(4)]
                for j in range(n_vec):
                    sl = pl.ds(j * _LANES, _LANES)
                    v = (bw[i, sl] + bp[i, sl]) + typev[tt, sl]
                    bw[i, sl] = v
                    k = j % 4
                    ss[k] = ss[k] + v
                    qq[k] = v * v + qq[k]
                s = (ss[0] + ss[1]) + (ss[2] + ss[3])
                q = (qq[0] + qq[1]) + (qq[2] + qq[3])
                mean = jnp.sum(s) * (1.0 / D)
                var = jnp.sum(q) * (1.0 / D) - mean * mean
                inv = _rsqrt_newton(var + _EPS)
                sinv[i] = inv
                snm2[i] = -(mean * inv)
                return 0

            lax.fori_loop(0, K, token_body, 0)

            def group_body(g, _):
                gb = g * _LANES
                invs = [sinv[gb + u] for u in range(_LANES)]
                nm2s = [snm2[gb + u] for u in range(_LANES)]
                for j in range(n_vec):
                    sl = pl.ds(j * _LANES, _LANES)
                    gam = gv[sl]
                    bet = bv[sl]
                    for u in range(_LANES):
                        normed = bw[gb + u, sl] * invs[u] + nm2s[u]
                        bw[gb + u, sl] = normed * gam + bet
                return 0

            lax.fori_loop(0, G, group_body, 0)
            off = pl.multiple_of(c * K, K)
            pltpu.async_copy(bw, out_hbm.at[pl.ds(base + off, K)], semW)

        fire(0, bwA, bpA, semA)

        def pair_body(c2, _):
            ca = 2 * c2

            @pl.when(c2 > 0)
            def _():
                wb_wait(bwB, semWB)

            fire(ca + 1, bwB, bpB, semB)
            drain(bwA, bpA, semA)
            compute_store(ca, bwA, bpA, semWA)
            drain(bwB, bpB, semB)

            @pl.when(ca + 2 < n_chunks)
            def _():
                wb_wait(bwA, semWA)
                fire(ca + 2, bwA, bpA, semA)

            compute_store(ca + 1, bwB, bpB, semWB)
            return 0

        lax.fori_loop(0, n_chunks // 2, pair_body, 0)
        wb_wait(bwA, semWA)
        wb_wait(bwB, semWB)

    out = emb_kernel(ids_w, ids_p, ids_t, word_table, pos_table, type_table,
                     gamma, beta)
    return out.reshape(B, S, D)


# speedup vs baseline: 1.6887x; 1.6887x over previous
import functools

import jax
import jax.numpy as jnp
from jax import lax
from jax.experimental import pallas as pl
from jax.experimental.pallas import tpu as pltpu
from jax.experimental.pallas import tpu_sc as plsc

_EPS = 1e-5
_LANES = 16


def _rsqrt_newton(x):
    i = lax.bitcast_convert_type(x, jnp.int32)
    i = jnp.int32(0x5F3759DF) - lax.shift_right_arithmetic(i, 1)
    y = lax.bitcast_convert_type(i, jnp.float32)
    for _ in range(4):
        y = y * (1.5 - 0.5 * x * y * y)
    return y


def kernel(input_ids, position_ids, token_type_ids, word_table, pos_table,
           type_table, gamma, beta):
    B, S = input_ids.shape
    V, D = word_table.shape
    N = B * S
    ids_w = input_ids.reshape(N).astype(jnp.int32)
    ids_p = position_ids.reshape(N).astype(jnp.int32)
    ids_t = token_type_ids.reshape(N).astype(jnp.int32)

    REP = 64
    T = type_table.shape[0]
    type_rep = jnp.tile(type_table, (REP, 1))

    info = plsc.get_sparse_core_info()
    NC, NS = info.num_cores, info.num_subcores
    NW = NC * NS
    K = 16
    per_w = N // NW
    n_chunks = per_w // K
    n_vec = D // _LANES

    mesh = plsc.VectorSubcoreMesh(core_axis_name="c", subcore_axis_name="s")

    @functools.partial(
        pl.kernel,
        mesh=mesh,
        out_type=jax.ShapeDtypeStruct((N, D), jnp.float32),
        compiler_params=pltpu.CompilerParams(needs_layout_passes=False),
        scratch_types=[
            pltpu.VMEM((per_w,), jnp.int32),
            pltpu.VMEM((per_w,), jnp.int32),
            pltpu.VMEM((per_w,), jnp.int32),
            pltpu.VMEM((K, D), jnp.float32),
            pltpu.VMEM((K, D), jnp.float32),
            pltpu.VMEM((K, D), jnp.float32),
            pltpu.VMEM((K, D), jnp.float32),
            pltpu.VMEM((K, D), jnp.float32),
            pltpu.VMEM((K, D), jnp.float32),
            pltpu.VMEM((D,), jnp.float32),
            pltpu.VMEM((D,), jnp.float32),
            pltpu.SMEM((K,), jnp.float32),
            pltpu.SMEM((K,), jnp.float32),
            pltpu.SemaphoreType.DMA,
            pltpu.SemaphoreType.DMA,
            pltpu.SemaphoreType.DMA,
            pltpu.SemaphoreType.DMA,
        ],
    )
    def emb_kernel(idsw_hbm, idsp_hbm, idst_hbm, word_hbm, pos_hbm, type_hbm,
                   gamma_hbm, beta_hbm, out_hbm,
                   idxw, idxp, idxt, bwA, bpA, btA, bwB, bpB, btB,
                   gv, bv, sinv, snm2, semA, semB, semWA, semWB):
        wid = lax.axis_index("s") * NC + lax.axis_index("c")
        base = wid * per_w
        pltpu.sync_copy(gamma_hbm, gv)
        pltpu.sync_copy(beta_hbm, bv)
        pltpu.sync_copy(idsw_hbm.at[pl.ds(base, per_w)], idxw)
        pltpu.sync_copy(idsp_hbm.at[pl.ds(base, per_w)], idxp)
        pltpu.sync_copy(idst_hbm.at[pl.ds(base, per_w)], idxt)

        iota = lax.iota(jnp.int32, _LANES)
        woff = wid * 29
        for g in range(per_w // _LANES):
            sl = pl.ds(g * _LANES, _LANES)
            k = lax.rem(woff + g * _LANES + iota, REP)
            idxt[sl] = idxt[sl] + T * k

        def fire(c, bw, bp, bt, sem):
            off = pl.multiple_of(c * K, K)
            pltpu.async_copy(word_hbm.at[idxw.at[pl.ds(off, K)]], bw, sem)
            pltpu.async_copy(pos_hbm.at[idxp.at[pl.ds(off, K)]], bp, sem)
            pltpu.async_copy(type_hbm.at[idxt.at[pl.ds(off, K)]], bt, sem)

        def drain(bw, bp, bt, sem):
            pltpu.make_async_copy(word_hbm.at[idxw.at[pl.ds(0, K)]], bw,
                                  sem).wait()
            pltpu.make_async_copy(pos_hbm.at[idxp.at[pl.ds(0, K)]], bp,
                                  sem).wait()
            pltpu.make_async_copy(type_hbm.at[idxt.at[pl.ds(0, K)]], bt,
                                  sem).wait()

        def wb_wait(bw, semW):
            pltpu.make_async_copy(bw, out_hbm.at[pl.ds(0, K)], semW).wait()

        def compute_store(c, bw, bp, bt, semW):
            def token_body(i, _):
                ss = [jnp.zeros((_LANES,), jnp.float32) for _ in range(4)]
                qq = [jnp.zeros((_LANES,), jnp.float32) for _ in range(4)]
                for j in range(n_vec):
                    sl = pl.ds(j * _LANES, _LANES)
                    v = (bw[i, sl] + bp[i, sl]) + bt[i, sl]
                    bw[i, sl] = v
                    k = j % 4
                    ss[k] = ss[k] + v
                    qq[k] = v * v + qq[k]
                s = (ss[0] + ss[1]) + (ss[2] + ss[3])
                q = (qq[0] + qq[1]) + (qq[2] + qq[3])
                mean = jnp.sum(s) * (1.0 / D)
                var = jnp.sum(q) * (1.0 / D) - mean * mean
                inv = _rsqrt_newton(var + _EPS)
                sinv[i] = inv
                snm2[i] = -(mean * inv)
                return 0

            lax.fori_loop(0, K, token_body, 0)

            invs = [sinv[i] for i in range(K)]
            nm2s = [snm2[i] for i in range(K)]
            for j in range(n_vec):
                sl = pl.ds(j * _LANES, _LANES)
                g = gv[sl]
                b = bv[sl]
                for i in range(K):
                    normed = bw[i, sl] * invs[i] + nm2s[i]
                    bw[i, sl] = normed * g + b

            off = pl.multiple_of(c * K, K)
            pltpu.async_copy(bw, out_hbm.at[pl.ds(base + off, K)], semW)

        fire(0, bwA, bpA, btA, semA)

        def pair_body(c2, _):
            ca = 2 * c2

            @pl.when(c2 > 0)
            def _():
                wb_wait(bwB, semWB)

            fire(ca + 1, bwB, bpB, btB, semB)
            drain(bwA, bpA, btA, semA)
            compute_store(ca, bwA, bpA, btA, semWA)
            drain(bwB, bpB, btB, semB)

            @pl.when(ca + 2 < n_chunks)
            def _():
                wb_wait(bwA, semWA)
                fire(ca + 2, bwA, bpA, btA, semA)

            compute_store(ca + 1, bwB, bpB, btB, semWB)
            return 0

        lax.fori_loop(0, n_chunks // 2, pair_body, 0)
        wb_wait(bwA, semWA)
        wb_wait(bwB, semWB)

    out = emb_kernel(ids_w, ids_p, ids_t, word_table, pos_table, type_rep,
                     gamma, beta)
    return out.reshape(B, S, D)
